# baseline (speedup 1.0000x reference)
import functools

import jax
import jax.numpy as jnp
from jax import lax
from jax.experimental import pallas as pl
from jax.experimental.pallas import tpu as pltpu
from jax.experimental.pallas import tpu_sc as plsc


@functools.lru_cache(maxsize=None)
def _make_gather_kernel(V, D, B):
    info = plsc.get_sparse_core_info()
    NW = info.num_cores * info.num_subcores
    assert B % NW == 0
    b_per_w = B // NW
    C = 1024
    assert b_per_w % C == 0
    n_iter = b_per_w // C

    mesh = plsc.VectorSubcoreMesh(core_axis_name="c", subcore_axis_name="s")

    @functools.partial(
        pl.kernel,
        mesh=mesh,
        out_type=jax.ShapeDtypeStruct((B, D), jnp.float32),
        scratch_types=[
            pltpu.VMEM((C,), jnp.int32),
            pltpu.VMEM((C, D), jnp.float32),
            pltpu.SemaphoreType.DMA,
        ],
        compiler_params=pltpu.CompilerParams(use_tc_tiling_on_sc=False),
    )
    def gather_kernel(idx_hbm, table_hbm, out_hbm, idx_v, rows_v, sem):
        wid = lax.axis_index("s") * info.num_cores + lax.axis_index("c")
        base = wid * b_per_w

        def body(i, carry):
            off = base + i * C
            pltpu.sync_copy(idx_hbm.at[pl.ds(off, C)], idx_v)
            pltpu.async_copy(table_hbm.at[idx_v], rows_v, sem).wait()
            pltpu.sync_copy(rows_v, out_hbm.at[pl.ds(off, C)])
            return carry

        lax.fori_loop(0, n_iter, body, 0)

    return gather_kernel


@jax.jit
def kernel(x, table):
    V, D = table.shape
    B = x.shape[0] * x.shape[1]
    flat_idx = x.reshape(B).astype(jnp.int32)
    out = _make_gather_kernel(V, D, B)(flat_idx, table)
    return out.reshape(x.shape + (D,))


# speedup vs baseline: 1.0149x; 1.0149x over previous
import functools

import jax
import jax.numpy as jnp
from jax import lax
from jax.experimental import pallas as pl
from jax.experimental.pallas import tpu as pltpu
from jax.experimental.pallas import tpu_sc as plsc


@functools.lru_cache(maxsize=None)
def _make_gather_kernel(V, D, B):
    info = plsc.get_sparse_core_info()
    NW = info.num_cores * info.num_subcores
    assert B % NW == 0
    b_per_w = B // NW
    C = 640
    assert b_per_w % C == 0
    n_iter = b_per_w // C
    assert n_iter % 2 == 0 and n_iter >= 4

    mesh = plsc.VectorSubcoreMesh(core_axis_name="c", subcore_axis_name="s")

    @functools.partial(
        pl.kernel,
        mesh=mesh,
        out_type=jax.ShapeDtypeStruct((B, D), jnp.float32),
        scratch_types=[
            pltpu.VMEM((b_per_w,), jnp.int32),
            pltpu.VMEM((C, D), jnp.float32),
            pltpu.VMEM((C, D), jnp.float32),
            pltpu.SemaphoreType.DMA,
            pltpu.SemaphoreType.DMA,
            pltpu.SemaphoreType.DMA,
            pltpu.SemaphoreType.DMA,
        ],
        compiler_params=pltpu.CompilerParams(use_tc_tiling_on_sc=False),
    )
    def gather_kernel(idx_hbm, table_hbm, out_hbm, idx_v, rows0, rows1,
                      gsem0, gsem1, ssem0, ssem1):
        wid = lax.axis_index("s") * info.num_cores + lax.axis_index("c")
        base = wid * b_per_w
        rows = (rows0, rows1)
        gsem = (gsem0, gsem1)
        ssem = (ssem0, ssem1)

        pltpu.sync_copy(idx_hbm.at[pl.ds(base, b_per_w)], idx_v)

        def start_gather(i, b):
            pltpu.async_copy(table_hbm.at[idx_v.at[pl.ds(i * C, C)]],
                             rows[b], gsem[b])

        def wait_gather(b):
            pltpu.make_async_copy(table_hbm.at[pl.ds(0, C)], rows[b],
                                  gsem[b]).wait()

        start_gather(0, 0)
        start_gather(1, 1)

        @pl.loop(0, n_iter - 2, step=2)
        def _steady(g):
            for b in range(2):
                i = g + b
                wait_gather(b)
                pltpu.async_copy(rows[b], out_hbm.at[pl.ds(base + i * C, C)],
                                 ssem[b])
                pltpu.make_async_copy(rows[b], out_hbm.at[pl.ds(base, C)],
                                      ssem[b]).wait()
                start_gather(i + 2, b)

        for b in range(2):
            i = n_iter - 2 + b
            wait_gather(b)
            pltpu.async_copy(rows[b], out_hbm.at[pl.ds(base + i * C, C)],
                             ssem[b])
        for b in range(2):
            pltpu.make_async_copy(rows[b], out_hbm.at[pl.ds(base, C)],
                                  ssem[b]).wait()

    return gather_kernel


@jax.jit
def kernel(x, table):
    V, D = table.shape
    B = x.shape[0] * x.shape[1]
    flat_idx = x.reshape(B).astype(jnp.int32)
    out = _make_gather_kernel(V, D, B)(flat_idx, table)
    return out.reshape(x.shape + (D,))


# speedup vs baseline: 1.0169x; 1.0020x over previous
import functools

import jax
import jax.numpy as jnp
from jax import lax
from jax.experimental import pallas as pl
from jax.experimental.pallas import tpu as pltpu
from jax.experimental.pallas import tpu_sc as plsc


@functools.lru_cache(maxsize=None)
def _make_gather_kernel(V, D, NB, T):
    info = plsc.get_sparse_core_info()
    NW = info.num_cores * info.num_subcores
    B = NB * T
    assert NB % NW == 0
    nb_per_w = NB // NW
    b_per_w = nb_per_w * T
    NBAT = 2
    C = NBAT * T
    assert nb_per_w % NBAT == 0
    n_iter = nb_per_w // NBAT
    assert n_iter % 2 == 0 and n_iter >= 4

    mesh = plsc.VectorSubcoreMesh(core_axis_name="c", subcore_axis_name="s")

    @functools.partial(
        pl.kernel,
        mesh=mesh,
        out_type=jax.ShapeDtypeStruct((NB, T, D), jnp.float32),
        scratch_types=[
            pltpu.VMEM((b_per_w,), jnp.int32),
            pltpu.VMEM((NBAT, T, D), jnp.float32),
            pltpu.VMEM((NBAT, T, D), jnp.float32),
            pltpu.SemaphoreType.DMA,
            pltpu.SemaphoreType.DMA,
            pltpu.SemaphoreType.DMA,
            pltpu.SemaphoreType.DMA,
        ],
        compiler_params=pltpu.CompilerParams(use_tc_tiling_on_sc=False),
    )
    def gather_kernel(idx_hbm, table_hbm, out_hbm, idx_v, rows0, rows1,
                      gsem0, gsem1, ssem0, ssem1):
        wid = lax.axis_index("s") * info.num_cores + lax.axis_index("c")
        base = wid * b_per_w
        bbase = wid * nb_per_w
        rows = (rows0, rows1)
        gsem = (gsem0, gsem1)
        ssem = (ssem0, ssem1)

        pltpu.sync_copy(idx_hbm.at[pl.ds(base, b_per_w)], idx_v)

        def start_gather(i, b):
            for j in range(NBAT):
                pltpu.async_copy(
                    table_hbm.at[idx_v.at[pl.ds(i * C + j * T, T)]],
                    rows[b].at[j], gsem[b])

        def wait_gather(b):
            for j in range(NBAT):
                pltpu.make_async_copy(table_hbm.at[pl.ds(0, T)],
                                      rows[b].at[j], gsem[b]).wait()

        def wait_store(b):
            pltpu.make_async_copy(rows[b], out_hbm.at[pl.ds(bbase, NBAT)],
                                  ssem[b]).wait()

        start_gather(0, 0)
        start_gather(1, 1)

        @pl.loop(0, n_iter - 2, step=2)
        def _steady(g):
            for b in range(2):
                i = g + b
                wait_gather(b)
                pltpu.async_copy(rows[b],
                                 out_hbm.at[pl.ds(bbase + i * NBAT, NBAT)],
                                 ssem[b])
                wait_store(b)
                start_gather(i + 2, b)

        for b in range(2):
            i = n_iter - 2 + b
            wait_gather(b)
            pltpu.async_copy(rows[b],
                             out_hbm.at[pl.ds(bbase + i * NBAT, NBAT)],
                             ssem[b])
        for b in range(2):
            wait_store(b)

    return gather_kernel


@jax.jit
def kernel(x, table):
    V, D = table.shape
    NB, T = x.shape
    flat_idx = x.reshape(NB * T).astype(jnp.int32)
    return _make_gather_kernel(V, D, NB, T)(flat_idx, table)
